# baseline (speedup 1.0000x reference)
import functools

import jax
import jax.numpy as jnp
from jax import lax
from jax.experimental import pallas as pl
from jax.experimental.pallas import tpu as pltpu
from jax.experimental.pallas import tpu_sc as plsc

B = 16384
DIM = 64
NC = 2
NS = 16
NW = NC * NS
BPW = B // NW
CHUNK = 128
NCHUNK = BPW // CHUNK


def _body(users, pos_items, neg_items, user_table, item_table,
          pos_out, neg_out,
          idx_u, idx_p, idx_n, u_rows, p_rows, n_rows,
          pos_v, neg_v, sem):
    wid = lax.axis_index("s") * NC + lax.axis_index("c")
    base = wid * BPW

    for j in range(NCHUNK):
        off = base + j * CHUNK
        pltpu.sync_copy(users.at[pl.ds(off, CHUNK)], idx_u.at[j])
        pltpu.sync_copy(pos_items.at[pl.ds(off, CHUNK)], idx_p.at[j])
        pltpu.sync_copy(neg_items.at[pl.ds(off, CHUNK)], idx_n.at[j])

    copies = []
    for j in range(NCHUNK):
        dst = pl.ds(j * CHUNK, CHUNK)
        copies.append(pltpu.make_async_copy(
            user_table.at[idx_u.at[j]], u_rows.at[dst], sem))
        copies.append(pltpu.make_async_copy(
            item_table.at[idx_p.at[j]], p_rows.at[dst], sem))
        copies.append(pltpu.make_async_copy(
            item_table.at[idx_n.at[j]], n_rows.at[dst], sem))
    for c in copies:
        c.start()
    for c in copies:
        c.wait()

    lane = lax.iota(jnp.int32, 16)
    perms = [lane ^ s for s in (8, 4, 2, 1)]

    gdn = lax.GatherDimensionNumbers(
        offset_dims=(), collapsed_slice_dims=(0,), start_index_map=(0,))

    def shuffle(v, p):
        return lax.gather(v, p[:, None], gdn, slice_sizes=(1,),
                          mode=lax.GatherScatterMode.PROMISE_IN_BOUNDS)

    def lanesum(v):
        for p in perms:
            v = v + shuffle(v, p)
        return v

    def group(g, _):
        b0 = g * 16
        accp_v = jnp.zeros((16,), jnp.float32)
        accn_v = jnp.zeros((16,), jnp.float32)
        for r in range(16):
            b = b0 + r
            u0 = u_rows[b, pl.ds(0, 16)]
            u1 = u_rows[b, pl.ds(16, 16)]
            u2 = u_rows[b, pl.ds(32, 16)]
            u3 = u_rows[b, pl.ds(48, 16)]
            p0 = p_rows[b, pl.ds(0, 16)]
            p1 = p_rows[b, pl.ds(16, 16)]
            p2 = p_rows[b, pl.ds(32, 16)]
            p3 = p_rows[b, pl.ds(48, 16)]
            n0 = n_rows[b, pl.ds(0, 16)]
            n1 = n_rows[b, pl.ds(16, 16)]
            n2 = n_rows[b, pl.ds(32, 16)]
            n3 = n_rows[b, pl.ds(48, 16)]
            accp = (u0 * p0 + u1 * p1) + (u2 * p2 + u3 * p3)
            accn = (u0 * n0 + u1 * n1) + (u2 * n2 + u3 * n3)
            sel = lane == r
            accp_v = jnp.where(sel, lanesum(accp), accp_v)
            accn_v = jnp.where(sel, lanesum(accn), accn_v)
        pos_v[pl.ds(b0, 16)] = accp_v
        neg_v[pl.ds(b0, 16)] = accn_v
        return 0

    lax.fori_loop(0, BPW // 16, group, 0)

    pltpu.sync_copy(pos_v, pos_out.at[pl.ds(base, BPW)])
    pltpu.sync_copy(neg_v, neg_out.at[pl.ds(base, BPW)])


@functools.partial(jax.jit, donate_argnums=())
def _run(users, pos_items, neg_items, user_table, item_table):
    mesh = plsc.VectorSubcoreMesh(core_axis_name="c", subcore_axis_name="s")
    f = pl.kernel(
        _body,
        out_type=(
            jax.ShapeDtypeStruct((B,), jnp.float32),
            jax.ShapeDtypeStruct((B,), jnp.float32),
        ),
        mesh=mesh,
        scratch_types=[
            pltpu.VMEM((NCHUNK, CHUNK), jnp.int32),
            pltpu.VMEM((NCHUNK, CHUNK), jnp.int32),
            pltpu.VMEM((NCHUNK, CHUNK), jnp.int32),
            pltpu.VMEM((BPW, DIM), jnp.float32),
            pltpu.VMEM((BPW, DIM), jnp.float32),
            pltpu.VMEM((BPW, DIM), jnp.float32),
            pltpu.VMEM((BPW,), jnp.float32),
            pltpu.VMEM((BPW,), jnp.float32),
            pltpu.SemaphoreType.DMA,
        ],
        compiler_params=pltpu.CompilerParams(use_tc_tiling_on_sc=False),
        name="basemf_sc_scores",
    )
    return f(users, pos_items, neg_items, user_table, item_table)


def kernel(users, pos_items, neg_items, user_table, item_table):
    return _run(users, pos_items, neg_items, user_table, item_table)
